# baseline (device time: 15594 ns/iter reference)
import jax
import jax.numpy as jnp
from jax import lax
from jax.experimental import pallas as pl
from jax.experimental.pallas import tpu as pltpu

N_DEV = 16
ROUND_XOR = (1, 3, 4, 8)
N_ROUNDS = len(ROUND_XOR)
P = 8


def _xor_for(k, p):
    return ROUND_XOR[(k + p) % N_ROUNDS]


def kernel(x):
    m, n = x.shape
    rows = m // P

    def body(x_ref, out_ref, comm_ref, send_sems, recv_sems):
        my = lax.axis_index("i")

        barrier_sem = pltpu.get_barrier_semaphore()
        for xr in ROUND_XOR:
            pl.semaphore_signal(
                barrier_sem,
                inc=1,
                device_id=(lax.bitwise_xor(my, xr),),
                device_id_type=pl.DeviceIdType.MESH,
            )
        out_ref[...] = x_ref[...].astype(jnp.bfloat16)
        pl.semaphore_wait(barrier_sem, N_ROUNDS)

        def make(k, p):
            partner = lax.bitwise_xor(my, _xor_for(k, p))
            return pltpu.make_async_remote_copy(
                src_ref=out_ref.at[pl.ds(p * rows, rows), :],
                dst_ref=comm_ref.at[k, pl.ds(p * rows, rows), :],
                send_sem=send_sems.at[k, p],
                recv_sem=recv_sems.at[k, p],
                device_id=(partner,),
                device_id_type=pl.DeviceIdType.MESH,
            )

        rdmas = {}
        for p in range(P):
            rdmas[(0, p)] = make(0, p)
            rdmas[(0, p)].start()
        for k in range(N_ROUNDS):
            for p in range(P):
                rdmas[(k, p)].wait()
                sl = pl.ds(p * rows, rows)
                out_ref[sl, :] = out_ref[sl, :] + comm_ref[k, sl, :]
                if k + 1 < N_ROUNDS:
                    rdmas[(k + 1, p)] = make(k + 1, p)
                    rdmas[(k + 1, p)].start()

    return pl.pallas_call(
        body,
        out_shape=jax.ShapeDtypeStruct((m, n), jnp.bfloat16),
        in_specs=[pl.BlockSpec(memory_space=pltpu.VMEM)],
        out_specs=pl.BlockSpec(memory_space=pltpu.VMEM),
        scratch_shapes=[
            pltpu.VMEM((N_ROUNDS, m, n), jnp.bfloat16),
            pltpu.SemaphoreType.DMA((N_ROUNDS, P)),
            pltpu.SemaphoreType.DMA((N_ROUNDS, P)),
        ],
        compiler_params=pltpu.CompilerParams(collective_id=0),
    )(x)


# device time: 14835 ns/iter; 1.0512x vs baseline; 1.0512x over previous
import jax
import jax.numpy as jnp
from jax import lax
from jax.experimental import pallas as pl
from jax.experimental.pallas import tpu as pltpu

N_DEV = 16
ROUND_XOR = (1, 3, 4, 8)
N_ROUNDS = len(ROUND_XOR)
P = 4


def _xor_for(k, p):
    return ROUND_XOR[(k + p) % N_ROUNDS]


def kernel(x):
    m, n = x.shape
    rows = m // P

    def body(x_ref, out_ref, comm_ref, send_sems, recv_sems):
        my = lax.axis_index("i")

        barrier_sem = pltpu.get_barrier_semaphore()
        for xr in ROUND_XOR:
            pl.semaphore_signal(
                barrier_sem,
                inc=1,
                device_id=(lax.bitwise_xor(my, xr),),
                device_id_type=pl.DeviceIdType.MESH,
            )
        out_ref[...] = x_ref[...].astype(jnp.bfloat16)
        pl.semaphore_wait(barrier_sem, N_ROUNDS)

        def make(k, p):
            partner = lax.bitwise_xor(my, _xor_for(k, p))
            return pltpu.make_async_remote_copy(
                src_ref=out_ref.at[pl.ds(p * rows, rows), :],
                dst_ref=comm_ref.at[k, pl.ds(p * rows, rows), :],
                send_sem=send_sems.at[k, p],
                recv_sem=recv_sems.at[k, p],
                device_id=(partner,),
                device_id_type=pl.DeviceIdType.MESH,
            )

        rdmas = {}
        for p in range(P):
            rdmas[(0, p)] = make(0, p)
            rdmas[(0, p)].start()
        for k in range(N_ROUNDS):
            for p in range(P):
                rdmas[(k, p)].wait()
                sl = pl.ds(p * rows, rows)
                out_ref[sl, :] = out_ref[sl, :] + comm_ref[k, sl, :]
                if k + 1 < N_ROUNDS:
                    rdmas[(k + 1, p)] = make(k + 1, p)
                    rdmas[(k + 1, p)].start()

    return pl.pallas_call(
        body,
        out_shape=jax.ShapeDtypeStruct((m, n), jnp.bfloat16),
        in_specs=[pl.BlockSpec(memory_space=pltpu.VMEM)],
        out_specs=pl.BlockSpec(memory_space=pltpu.VMEM),
        scratch_shapes=[
            pltpu.VMEM((N_ROUNDS, m, n), jnp.bfloat16),
            pltpu.SemaphoreType.DMA((N_ROUNDS, P)),
            pltpu.SemaphoreType.DMA((N_ROUNDS, P)),
        ],
        compiler_params=pltpu.CompilerParams(collective_id=0),
    )(x)
